# baseline (speedup 1.0000x reference)
import functools

import jax
import jax.numpy as jnp
from jax import lax
from jax.experimental import pallas as pl
from jax.experimental.pallas import tpu as pltpu
from jax.experimental.pallas import tpu_sc as plsc

_LANES = 16
_CHUNK = 800
_K = 5
_SUB = _CHUNK // _K


def _make_sc_kernel(S, N, d, n_workers):
    rows_per_w = S // n_workers
    n_chunks = N // _CHUNK
    inner_iters = _CHUNK // _LANES
    total_chunks = rows_per_w * n_chunks

    mesh = plsc.VectorSubcoreMesh(
        core_axis_name="c", subcore_axis_name="s", num_cores=1
    )

    @functools.partial(
        pl.kernel,
        out_type=jax.ShapeDtypeStruct((S * d,), jnp.float32),
        mesh=mesh,
        scratch_types=[
            pltpu.VMEM((_CHUNK,), jnp.float32),
            pltpu.VMEM((_CHUNK,), jnp.float32),
            pltpu.VMEM((8, d), jnp.float32),
            pltpu.VMEM((d,), jnp.float32),
            pltpu.SemaphoreType.DMA,
            pltpu.SemaphoreType.DMA,
        ],
    )
    def scan_argmax_gather(
        smp_hbm, x_hbm, out_hbm, buf0, buf1, xbuf, row_v, sem0, sem1
    ):
        cid = lax.axis_index("c")
        sid = lax.axis_index("s")
        wid = sid * 2 + cid
        bufs = (buf0, buf1)
        sems = (sem0, sem1)
        iota = lax.iota(jnp.int32, _LANES)

        def start_chunk(t):
            row = wid * rows_per_w + (t // n_chunks)
            off = pl.multiple_of(row * N + (t % n_chunks) * _CHUNK, 8)
            return pltpu.async_copy(
                smp_hbm.at[pl.ds(off, _CHUNK)], bufs[t % 2], sems[t % 2]
            )

        descs = [None] * total_chunks
        descs[0] = start_chunk(0)
        skip_dma = True
        gm = None
        gidx = None
        row_best = []
        neg_inf = jnp.full((_LANES,), -jnp.inf, jnp.float32)
        zeros_i = jnp.zeros((_LANES,), jnp.int32)
        for t in range(total_chunks):
            ci = t % n_chunks
            if t + 1 < total_chunks and (t + 1) % n_chunks == 0:
                descs[t + 1] = start_chunk(t + 1)
            if ci == 0:
                descs[t].wait()
            if ci == 0:
                gm = neg_inf
                gidx = zeros_i
            buf = bufs[t % 2]
            base = ci * _CHUNK

            def body(i, carry, buf=buf):
                ms, ss = carry
                st = jnp.full((_LANES,), i, jnp.int32)
                nm = []
                ns = []
                for k in range(_K):
                    v = buf[pl.ds(k * _SUB + i * _LANES, _LANES)]
                    p = v > ms[k]
                    nm.append(jnp.where(p, v, ms[k]))
                    ns.append(jnp.where(p, st, ss[k]))
                return tuple(nm), tuple(ns)

            init = ((neg_inf,) * _K, (zeros_i,) * _K)
            if ci == 0:
                ms, ss = plsc.parallel_loop(0, _SUB // _LANES, carry=init, unroll=2)(
                    body
                )

                for k in range(_K):
                    idx = ss[k] * _LANES + iota + (base + k * _SUB)
                    takev = (ms[k] > gm) | ((ms[k] == gm) & (idx < gidx))
                    gm = jnp.where(takev, ms[k], gm)
                    gidx = jnp.where(takev, idx, gidx)

            if ci == n_chunks - 1:
                best_v = gm[0]
                best = gidx[0]
                for j in range(1, _LANES):
                    v = gm[j]
                    ij = gidx[j]
                    take = (v > best_v) | ((v == best_v) & (ij < best))
                    best_v = jnp.where(take, v, best_v)
                    best = jnp.where(take, ij, best)
                row_best.append(best)

        for r in range(rows_per_w):
            best = row_best[r]
            base8 = pl.multiple_of((best // 8) * 8, 8)
            pltpu.sync_copy(x_hbm.at[pl.ds(base8, 8)], xbuf)
            rr = best - base8
            for k in range(d // _LANES):
                row_v[pl.ds(k * _LANES, _LANES)] = xbuf[rr, pl.ds(k * _LANES, _LANES)]
            row = wid * rows_per_w + r
            pltpu.sync_copy(row_v, out_hbm.at[pl.ds(pl.multiple_of(row * d, 8), d)])

    return scan_argmax_gather


def kernel(X, samples, num_samples):
    S, N, _ = samples.shape
    d = X.shape[1]
    info = plsc.get_sparse_core_info()
    n_workers = info.num_cores * info.num_subcores
    sc_fn = _make_sc_kernel(S, N, d, n_workers)
    return sc_fn(samples.reshape(S * N), X).reshape(S, d)


# speedup vs baseline: 1.0507x; 1.0507x over previous
import functools

import jax
import jax.numpy as jnp
from jax import lax
from jax.experimental import pallas as pl
from jax.experimental.pallas import tpu as pltpu
from jax.experimental.pallas import tpu_sc as plsc

_LANES = 16
_CHUNK = 20000
_K = 5
_SUB = _CHUNK // _K


def _make_argmax_kernel(S, N, n_workers):
    rows_per_w = S // n_workers
    n_chunks = N // _CHUNK
    total_chunks = rows_per_w * n_chunks

    mesh = plsc.VectorSubcoreMesh(core_axis_name="c", subcore_axis_name="s")

    @functools.partial(
        pl.kernel,
        out_type=jax.ShapeDtypeStruct((n_workers, _LANES), jnp.int32),
        mesh=mesh,
        compiler_params=pltpu.CompilerParams(use_tc_tiling_on_sc=False),
        scratch_types=[
            pltpu.VMEM((_CHUNK,), jnp.float32),
            pltpu.VMEM((_CHUNK,), jnp.float32),
            pltpu.VMEM((1, _LANES), jnp.int32),
            pltpu.SemaphoreType.DMA,
            pltpu.SemaphoreType.DMA,
        ],
    )
    def scan_argmax(obj_hbm, idcs_hbm, buf0, buf1, ibuf, sem0, sem1):
        cid = lax.axis_index("c")
        sid = lax.axis_index("s")
        wid = sid * 2 + cid
        bufs = (buf0, buf1)
        sems = (sem0, sem1)
        iota = lax.iota(jnp.int32, _LANES)

        def start_chunk(t):
            row = wid * rows_per_w + (t // n_chunks)
            off = (t % n_chunks) * _CHUNK
            return pltpu.async_copy(
                obj_hbm.at[row, pl.ds(off, _CHUNK)], bufs[t % 2], sems[t % 2]
            )

        descs = [None] * total_chunks
        descs[0] = start_chunk(0)
        gm = None
        gidx = None
        row_best = []
        neg_inf = jnp.full((_LANES,), -jnp.inf, jnp.float32)
        zeros_i = jnp.zeros((_LANES,), jnp.int32)
        for t in range(total_chunks):
            ci = t % n_chunks
            if t + 1 < total_chunks:
                descs[t + 1] = start_chunk(t + 1)
            descs[t].wait()
            if ci == 0:
                gm = neg_inf
                gidx = zeros_i
            buf = bufs[t % 2]
            base = ci * _CHUNK

            def body(i, carry, buf=buf):
                ms, ss = carry
                st = jnp.full((_LANES,), i, jnp.int32)
                nm = []
                ns = []
                for k in range(_K):
                    v = buf[pl.ds(k * _SUB + i * _LANES, _LANES)]
                    p = v > ms[k]
                    nm.append(jnp.where(p, v, ms[k]))
                    ns.append(jnp.where(p, st, ss[k]))
                return tuple(nm), tuple(ns)

            init = ((neg_inf,) * _K, (zeros_i,) * _K)
            ms, ss = plsc.parallel_loop(0, _SUB // _LANES, carry=init, unroll=2)(body)

            for k in range(_K):
                idx = ss[k] * _LANES + iota + (base + k * _SUB)
                takev = (ms[k] > gm) | ((ms[k] == gm) & (idx < gidx))
                gm = jnp.where(takev, ms[k], gm)
                gidx = jnp.where(takev, idx, gidx)

            if ci == n_chunks - 1:
                best_v = gm[0]
                best = gidx[0]
                for j in range(1, _LANES):
                    v = gm[j]
                    ij = gidx[j]
                    take = (v > best_v) | ((v == best_v) & (ij < best))
                    best_v = jnp.where(take, v, best_v)
                    best = jnp.where(take, ij, best)
                row_best.append(best)

        vout = jnp.zeros((_LANES,), jnp.int32)
        for r in range(rows_per_w):
            vout = jnp.where(iota == r, row_best[r], vout)
        ibuf.at[0][...] = vout
        pltpu.sync_copy(ibuf, idcs_hbm.at[pl.ds(wid, 1)])

    return scan_argmax


def _make_gather_kernel(S, N, d, n_workers):
    rows_per_w = S // n_workers
    mesh = plsc.VectorSubcoreMesh(core_axis_name="c", subcore_axis_name="s")

    @functools.partial(
        pl.kernel,
        out_type=jax.ShapeDtypeStruct((S * d,), jnp.float32),
        mesh=mesh,
        compiler_params=pltpu.CompilerParams(needs_layout_passes=False),
        scratch_types=[
            pltpu.VMEM((n_workers, _LANES), jnp.int32),
            pltpu.VMEM((d, 128), jnp.float32),
            pltpu.VMEM((d,), jnp.float32),
        ],
    )
    def gather_rows(xt_hbm, idcs_hbm, out_hbm, ibuf, xblk, row_v):
        cid = lax.axis_index("c")
        sid = lax.axis_index("s")
        wid = sid * 2 + cid
        iota = lax.iota(jnp.int32, _LANES)
        zeros = jnp.zeros((_LANES,), jnp.int32)

        pltpu.sync_copy(idcs_hbm, ibuf)
        mine = plsc.load_gather(ibuf, [jnp.full((_LANES,), wid, jnp.int32), iota])
        for r in range(rows_per_w):
            best = mine[r]
            base = pl.multiple_of((best // 128) * 128, 128)
            col = best - base
            pltpu.sync_copy(xt_hbm.at[pl.ds(0, d), pl.ds(base, 128)], xblk)
            colv = zeros + col
            for k in range(d // _LANES):
                vals = plsc.load_gather(xblk, [iota + k * _LANES, colv])
                row_v[pl.ds(k * _LANES, _LANES)] = vals
            row = wid * rows_per_w + r
            pltpu.sync_copy(
                row_v, out_hbm.at[pl.ds(pl.multiple_of(row * d, 8), d)]
            )

    return gather_rows


def kernel(X, samples, num_samples):
    S, N, _ = samples.shape
    d = X.shape[1]
    info = plsc.get_sparse_core_info()
    n_workers = info.num_cores * info.num_subcores
    obj = jnp.squeeze(samples, axis=-1)
    xt = jnp.transpose(X)
    idcs = _make_argmax_kernel(S, N, n_workers)(obj)
    out = _make_gather_kernel(S, N, d, n_workers)(xt, idcs)
    return out.reshape(S, d)


# speedup vs baseline: 1.0524x; 1.0016x over previous
import functools

import jax
import jax.numpy as jnp
from jax import lax
from jax.experimental import pallas as pl
from jax.experimental.pallas import tpu as pltpu
from jax.experimental.pallas import tpu_sc as plsc

_LANES = 16
_CHUNK = 20000
_K = 5
_SUB = _CHUNK // _K


def _make_argmax_kernel(S, N, n_workers):
    rows_per_w = S // n_workers
    n_chunks = N // _CHUNK
    total_chunks = rows_per_w * n_chunks

    mesh = plsc.VectorSubcoreMesh(core_axis_name="c", subcore_axis_name="s")

    @functools.partial(
        pl.kernel,
        out_type=jax.ShapeDtypeStruct((n_workers, _LANES), jnp.int32),
        mesh=mesh,
        compiler_params=pltpu.CompilerParams(use_tc_tiling_on_sc=False),
        scratch_types=[
            pltpu.VMEM((_CHUNK,), jnp.float32),
            pltpu.VMEM((_CHUNK,), jnp.float32),
            pltpu.VMEM((1, _LANES), jnp.int32),
            pltpu.SemaphoreType.DMA,
            pltpu.SemaphoreType.DMA,
        ],
    )
    def scan_argmax(obj_hbm, idcs_hbm, buf0, buf1, ibuf, sem0, sem1):
        cid = lax.axis_index("c")
        sid = lax.axis_index("s")
        wid = sid * 2 + cid
        bufs = (buf0, buf1)
        sems = (sem0, sem1)
        iota = lax.iota(jnp.int32, _LANES)

        def start_chunk(t):
            row = wid * rows_per_w + (t // n_chunks)
            off = (t % n_chunks) * _CHUNK
            return pltpu.async_copy(
                obj_hbm.at[row, pl.ds(off, _CHUNK)], bufs[t % 2], sems[t % 2]
            )

        descs = [None] * total_chunks
        descs[0] = start_chunk(0)
        gm = None
        gidx = None
        row_best = []
        neg_inf = jnp.full((_LANES,), -jnp.inf, jnp.float32)
        zeros_i = jnp.zeros((_LANES,), jnp.int32)
        for t in range(total_chunks):
            ci = t % n_chunks
            if t + 1 < total_chunks:
                descs[t + 1] = start_chunk(t + 1)
            descs[t].wait()
            if ci == 0:
                gm = neg_inf
                gidx = zeros_i
            buf = bufs[t % 2]
            base = ci * _CHUNK

            def body(i, carry, buf=buf):
                ms, ss = carry
                st = jnp.full((_LANES,), i, jnp.int32)
                nm = []
                ns = []
                for k in range(_K):
                    v = buf[pl.ds(k * _SUB + i * _LANES, _LANES)]
                    p = v > ms[k]
                    nm.append(jnp.where(p, v, ms[k]))
                    ns.append(jnp.where(p, st, ss[k]))
                return tuple(nm), tuple(ns)

            init = ((neg_inf,) * _K, (zeros_i,) * _K)
            ms, ss = plsc.parallel_loop(0, _SUB // _LANES, carry=init, unroll=2)(body)

            for k in range(_K):
                idx = ss[k] * _LANES + iota + (base + k * _SUB)
                takev = (ms[k] > gm) | ((ms[k] == gm) & (idx < gidx))
                gm = jnp.where(takev, ms[k], gm)
                gidx = jnp.where(takev, idx, gidx)

            if ci == n_chunks - 1:
                best_v = gm[0]
                best = gidx[0]
                for j in range(1, _LANES):
                    v = gm[j]
                    ij = gidx[j]
                    take = (v > best_v) | ((v == best_v) & (ij < best))
                    best_v = jnp.where(take, v, best_v)
                    best = jnp.where(take, ij, best)
                row_best.append(best)

        vout = jnp.zeros((_LANES,), jnp.int32)
        for r in range(rows_per_w):
            vout = jnp.where(iota == r, row_best[r], vout)
        ibuf.at[0][...] = vout
        pltpu.sync_copy(ibuf, idcs_hbm.at[pl.ds(wid, 1)])

    return scan_argmax


def _make_gather_kernel(S, N, d, n_workers):
    rows_per_w = S // n_workers
    mesh = plsc.VectorSubcoreMesh(core_axis_name="c", subcore_axis_name="s")

    @functools.partial(
        pl.kernel,
        out_type=jax.ShapeDtypeStruct((S * d,), jnp.float32),
        mesh=mesh,
        compiler_params=pltpu.CompilerParams(needs_layout_passes=False),
        scratch_types=[
            pltpu.VMEM((n_workers, _LANES), jnp.int32),
            pltpu.VMEM((d, 128), jnp.float32),
            pltpu.VMEM((d,), jnp.float32),
        ],
    )
    def gather_rows(xt_hbm, idcs_hbm, out_hbm, ibuf, xblk, row_v):
        cid = lax.axis_index("c")
        sid = lax.axis_index("s")
        wid = sid * 2 + cid
        iota = lax.iota(jnp.int32, _LANES)
        zeros = jnp.zeros((_LANES,), jnp.int32)

        pltpu.sync_copy(idcs_hbm, ibuf)
        mine = plsc.load_gather(ibuf, [jnp.full((_LANES,), wid, jnp.int32), iota])
        for r in range(rows_per_w):
            best = mine[r]
            base = pl.multiple_of((best // 128) * 128, 128)
            col = best - base
            pltpu.sync_copy(xt_hbm.at[pl.ds(0, d), pl.ds(base, 128)], xblk)
            colv = zeros + col
            for k in range(d // _LANES):
                vals = plsc.load_gather(xblk, [iota + k * _LANES, colv])
                row_v[pl.ds(k * _LANES, _LANES)] = vals
            row = wid * rows_per_w + r
            pltpu.sync_copy(
                row_v, out_hbm.at[pl.ds(pl.multiple_of(row * d, 8), d)]
            )

    return gather_rows


def kernel(X, samples, num_samples):
    S, N, _ = samples.shape
    d = X.shape[1]
    info = plsc.get_sparse_core_info()
    n_workers = info.num_cores * info.num_subcores
    obj = samples[:, :, 0]
    xt = jnp.transpose(X)
    idcs = _make_argmax_kernel(S, N, n_workers)(obj)
    out = _make_gather_kernel(S, N, d, n_workers)(xt, idcs)
    return out.reshape(S, d)


# speedup vs baseline: 4.7139x; 4.4793x over previous
import functools

import jax
import jax.numpy as jnp
from jax import lax
from jax.experimental import pallas as pl
from jax.experimental.pallas import tpu as pltpu
from jax.experimental.pallas import tpu_sc as plsc

_LANES = 16
_GROUP = 8
_WPG = 4
_SPAN = 24960
_WIDTHS = (6144, 6144, 6144, 6528)
_PREFIX = (0, 6144, 12288, 18432)
_BUF_W = max(_WIDTHS)
_TAIL = 160
_TAIL_OFF = _WPG * _SPAN


def _lex_merge(bm, bi, vm, vi):
    take = (vm > bm) | ((vm == bm) & (vi < bi))
    return jnp.where(take, vm, bm), jnp.where(take, vi, bi)


def _make_argmax_kernel(S, N, n_workers):
    n_groups = S // _GROUP
    mesh = plsc.VectorSubcoreMesh(core_axis_name="c", subcore_axis_name="s")

    @functools.partial(
        pl.kernel,
        out_type=jax.ShapeDtypeStruct((S * _LANES // 2,), jnp.int32),
        mesh=mesh,
        scratch_types=[
            pltpu.VMEM((_GROUP, _BUF_W), jnp.float32),
            pltpu.VMEM((_GROUP, _BUF_W), jnp.float32),
            pltpu.VMEM((_GROUP * _LANES,), jnp.float32),
            pltpu.VMEM((_GROUP * _LANES,), jnp.int32),
            pltpu.VMEM((_WPG * _GROUP * _LANES,), jnp.float32),
            pltpu.VMEM((_WPG * _GROUP * _LANES,), jnp.int32),
            pltpu.VMEM((_LANES,), jnp.int32),
            pltpu.VMEM_SHARED((16 * _GROUP * _LANES,), jnp.float32),
            pltpu.VMEM_SHARED((16 * _GROUP * _LANES,), jnp.int32),
            pltpu.SemaphoreType.DMA,
            pltpu.SemaphoreType.DMA,
        ],
    )
    def scan_argmax(
        obj_hbm,
        tail_hbm,
        idcs_hbm,
        buf0,
        buf1,
        mst,
        ist,
        gbufm,
        gbufi,
        obuf,
        stage_m,
        stage_i,
        sem0,
        sem1,
    ):
        cid = lax.axis_index("c")
        sid = lax.axis_index("s")
        gsc = sid // _WPG
        q = sid % _WPG
        row8 = pl.multiple_of((cid * (n_groups // 2) + gsc) * _GROUP, 8)
        start = q * _SPAN
        bufs = (buf0, buf1)
        sems = (sem0, sem1)
        iota = lax.iota(jnp.int32, _LANES)
        neg_inf = jnp.full((_LANES,), -jnp.inf, jnp.float32)
        zeros_i = jnp.zeros((_LANES,), jnp.int32)

        def start_chunk(c):
            off = pl.multiple_of(start + _PREFIX[c], 128)
            w = _WIDTHS[c]
            return pltpu.async_copy(
                obj_hbm.at[pl.ds(row8, _GROUP), pl.ds(off, w)],
                bufs[c % 2].at[:, pl.ds(0, w)],
                sems[c % 2],
            )

        def scan_chunk(buf, width, col_base, gms, gidxs):
            def body(i, carry):
                ms, ss = carry
                st = jnp.full((_LANES,), i, jnp.int32)
                nm = []
                ns = []
                for k in range(_GROUP):
                    v = buf[k, pl.ds(i * _LANES, _LANES)]
                    p = v > ms[k]
                    nm.append(jnp.where(p, v, ms[k]))
                    ns.append(jnp.where(p, st, ss[k]))
                return tuple(nm), tuple(ns)

            init = ((neg_inf,) * _GROUP, (zeros_i,) * _GROUP)
            ms, ss = plsc.parallel_loop(0, width // _LANES, carry=init, unroll=2)(
                body
            )
            out_m = []
            out_i = []
            for k in range(_GROUP):
                idx = ss[k] * _LANES + iota + col_base
                m2, i2 = _lex_merge(gms[k], gidxs[k], ms[k], idx)
                out_m.append(m2)
                out_i.append(i2)
            return out_m, out_i

        gms = [neg_inf] * _GROUP
        gidxs = [zeros_i] * _GROUP
        descs = [None] * len(_WIDTHS)
        descs[0] = start_chunk(0)
        for c in range(len(_WIDTHS)):
            if c + 1 < len(_WIDTHS):
                descs[c + 1] = start_chunk(c + 1)
            descs[c].wait()
            gms, gidxs = scan_chunk(
                bufs[c % 2], _WIDTHS[c], start + _PREFIX[c], gms, gidxs
            )

        pltpu.sync_copy(
            tail_hbm.at[pl.ds(row8, _GROUP)], buf0.at[:, pl.ds(0, 256)]
        )
        gms, gidxs = scan_chunk(buf0, _TAIL, _TAIL_OFF, gms, gidxs)

        for k in range(_GROUP):
            mst[pl.ds(k * _LANES, _LANES)] = gms[k]
            ist[pl.ds(k * _LANES, _LANES)] = gidxs[k]
        sz = _GROUP * _LANES
        dst0 = pl.multiple_of(sid * sz, 8)
        pltpu.sync_copy(mst, stage_m.at[pl.ds(dst0, sz)])
        pltpu.sync_copy(ist, stage_i.at[pl.ds(dst0, sz)])
        plsc.subcore_barrier()

        gsz = _WPG * sz
        src0 = pl.multiple_of(gsc * gsz, 8)
        pltpu.sync_copy(stage_m.at[pl.ds(src0, gsz)], gbufm)
        pltpu.sync_copy(stage_i.at[pl.ds(src0, gsz)], gbufi)

        bests = []
        for r in range(2):
            k = q * 2 + r
            koff = k * _LANES
            bm = gbufm[pl.ds(koff, _LANES)]
            bi = gbufi[pl.ds(koff, _LANES)]
            for j in range(1, _WPG):
                vm = gbufm[pl.ds(j * sz + koff, _LANES)]
                vi = gbufi[pl.ds(j * sz + koff, _LANES)]
                bm, bi = _lex_merge(bm, bi, vm, vi)
            best_v = bm[0]
            best = bi[0]
            for j in range(1, _LANES):
                v = bm[j]
                ij = bi[j]
                take = (v > best_v) | ((v == best_v) & (ij < best))
                best_v = jnp.where(take, v, best_v)
                best = jnp.where(take, ij, best)
            bests.append(best)
        vout = jnp.where(iota == 0, bests[0], jnp.where(iota == 1, bests[1], 0))
        obuf[...] = vout
        slot = (row8 + q * 2) // 2
        dst = pl.multiple_of(slot * _LANES, 8)
        pltpu.sync_copy(obuf, idcs_hbm.at[pl.ds(dst, _LANES)])

    return scan_argmax


def _make_gather_kernel(S, N, d, n_workers):
    rows_per_w = S // n_workers
    mesh = plsc.VectorSubcoreMesh(core_axis_name="c", subcore_axis_name="s")

    @functools.partial(
        pl.kernel,
        out_type=jax.ShapeDtypeStruct((S * d,), jnp.float32),
        mesh=mesh,
        compiler_params=pltpu.CompilerParams(needs_layout_passes=False),
        scratch_types=[
            pltpu.VMEM((S * _LANES // 2,), jnp.int32),
            pltpu.VMEM((d, 128), jnp.float32),
            pltpu.VMEM((d,), jnp.float32),
        ],
    )
    def gather_rows(xt_hbm, idcs_hbm, out_hbm, ibuf, xblk, row_v):
        cid = lax.axis_index("c")
        sid = lax.axis_index("s")
        wid = sid * 2 + cid
        iota = lax.iota(jnp.int32, _LANES)

        pltpu.sync_copy(idcs_hbm, ibuf)
        lane0 = ibuf[pl.ds(wid * _LANES, _LANES)]
        for r in range(rows_per_w):
            best = lane0[r]
            base = pl.multiple_of((best // 128) * 128, 128)
            col = best - base
            pltpu.sync_copy(xt_hbm.at[pl.ds(0, d), pl.ds(base, 128)], xblk)
            colv = jnp.zeros((_LANES,), jnp.int32) + col
            for k in range(d // _LANES):
                vals = plsc.load_gather(xblk, [iota + k * _LANES, colv])
                row_v[pl.ds(k * _LANES, _LANES)] = vals
            row = wid * rows_per_w + r
            pltpu.sync_copy(
                row_v, out_hbm.at[pl.ds(pl.multiple_of(row * d, 8), d)]
            )

    return gather_rows


def kernel(X, samples, num_samples):
    S, N, _ = samples.shape
    d = X.shape[1]
    info = plsc.get_sparse_core_info()
    n_workers = info.num_cores * info.num_subcores
    obj = jnp.squeeze(samples, axis=-1)
    xt = jnp.transpose(X)
    tail = jnp.pad(
        lax.slice(obj, (0, _TAIL_OFF), (S, N)), ((0, 0), (0, 256 - _TAIL))
    )
    idcs = _make_argmax_kernel(S, N, n_workers)(obj, tail)
    out = _make_gather_kernel(S, N, d, n_workers)(xt, idcs)
    return out.reshape(S, d)


# speedup vs baseline: 4.7139x; 1.0000x over previous
import functools

import jax
import jax.numpy as jnp
from jax import lax
from jax.experimental import pallas as pl
from jax.experimental.pallas import tpu as pltpu
from jax.experimental.pallas import tpu_sc as plsc

_LANES = 16
_GROUP = 8
_WPG = 4
_SPAN = 24960
_WIDTHS = (6144, 6144, 6144, 6528)
_PREFIX = (0, 6144, 12288, 18432)
_BUF_W = max(_WIDTHS)
_TAIL = 160
_TAIL_OFF = _WPG * _SPAN


def _lex_merge(bm, bi, vm, vi):
    take = (vm > bm) | ((vm == bm) & (vi < bi))
    return jnp.where(take, vm, bm), jnp.where(take, vi, bi)


def _make_argmax_kernel(S, N, n_workers):
    n_groups = S // _GROUP
    mesh = plsc.VectorSubcoreMesh(core_axis_name="c", subcore_axis_name="s")

    @functools.partial(
        pl.kernel,
        out_type=jax.ShapeDtypeStruct((S * _LANES // 2,), jnp.int32),
        mesh=mesh,
        scratch_types=[
            pltpu.VMEM((_GROUP, _BUF_W), jnp.float32),
            pltpu.VMEM((_GROUP, _BUF_W), jnp.float32),
            pltpu.VMEM((_GROUP * _LANES,), jnp.float32),
            pltpu.VMEM((_GROUP * _LANES,), jnp.int32),
            pltpu.VMEM((_WPG * _GROUP * _LANES,), jnp.float32),
            pltpu.VMEM((_WPG * _GROUP * _LANES,), jnp.int32),
            pltpu.VMEM((_LANES,), jnp.int32),
            pltpu.VMEM_SHARED((16 * _GROUP * _LANES,), jnp.float32),
            pltpu.VMEM_SHARED((16 * _GROUP * _LANES,), jnp.int32),
            pltpu.SemaphoreType.DMA,
            pltpu.SemaphoreType.DMA,
        ],
    )
    def scan_argmax(
        obj_hbm,
        tail_hbm,
        idcs_hbm,
        buf0,
        buf1,
        mst,
        ist,
        gbufm,
        gbufi,
        obuf,
        stage_m,
        stage_i,
        sem0,
        sem1,
    ):
        cid = lax.axis_index("c")
        sid = lax.axis_index("s")
        gsc = sid // _WPG
        q = sid % _WPG
        row8 = pl.multiple_of((cid * (n_groups // 2) + gsc) * _GROUP, 8)
        start = q * _SPAN
        bufs = (buf0, buf1)
        sems = (sem0, sem1)
        iota = lax.iota(jnp.int32, _LANES)
        neg_inf = jnp.full((_LANES,), -jnp.inf, jnp.float32)
        zeros_i = jnp.zeros((_LANES,), jnp.int32)

        def start_chunk(c):
            off = pl.multiple_of(start + _PREFIX[c], 128)
            w = _WIDTHS[c]
            return pltpu.async_copy(
                obj_hbm.at[pl.ds(row8, _GROUP), pl.ds(off, w)],
                bufs[c % 2].at[:, pl.ds(0, w)],
                sems[c % 2],
            )

        def scan_chunk(buf, width, col_base, gms, gidxs):
            def body(i, carry):
                ms, ss = carry
                st = jnp.full((_LANES,), i, jnp.int32)
                nm = []
                ns = []
                for k in range(_GROUP):
                    v = buf[k, pl.ds(i * _LANES, _LANES)]
                    p = v > ms[k]
                    nm.append(jnp.where(p, v, ms[k]))
                    ns.append(jnp.where(p, st, ss[k]))
                return tuple(nm), tuple(ns)

            init = ((neg_inf,) * _GROUP, (zeros_i,) * _GROUP)
            ms, ss = plsc.parallel_loop(0, width // _LANES, carry=init, unroll=4)(
                body
            )
            out_m = []
            out_i = []
            for k in range(_GROUP):
                idx = ss[k] * _LANES + iota + col_base
                m2, i2 = _lex_merge(gms[k], gidxs[k], ms[k], idx)
                out_m.append(m2)
                out_i.append(i2)
            return out_m, out_i

        gms = [neg_inf] * _GROUP
        gidxs = [zeros_i] * _GROUP
        descs = [None] * len(_WIDTHS)
        descs[0] = start_chunk(0)
        for c in range(len(_WIDTHS)):
            if c + 1 < len(_WIDTHS):
                descs[c + 1] = start_chunk(c + 1)
            descs[c].wait()
            gms, gidxs = scan_chunk(
                bufs[c % 2], _WIDTHS[c], start + _PREFIX[c], gms, gidxs
            )

        pltpu.sync_copy(
            tail_hbm.at[pl.ds(row8, _GROUP)], buf0.at[:, pl.ds(0, 256)]
        )
        gms, gidxs = scan_chunk(buf0, _TAIL, _TAIL_OFF, gms, gidxs)

        for k in range(_GROUP):
            mst[pl.ds(k * _LANES, _LANES)] = gms[k]
            ist[pl.ds(k * _LANES, _LANES)] = gidxs[k]
        sz = _GROUP * _LANES
        dst0 = pl.multiple_of(sid * sz, 8)
        pltpu.sync_copy(mst, stage_m.at[pl.ds(dst0, sz)])
        pltpu.sync_copy(ist, stage_i.at[pl.ds(dst0, sz)])
        plsc.subcore_barrier()

        gsz = _WPG * sz
        src0 = pl.multiple_of(gsc * gsz, 8)
        pltpu.sync_copy(stage_m.at[pl.ds(src0, gsz)], gbufm)
        pltpu.sync_copy(stage_i.at[pl.ds(src0, gsz)], gbufi)

        bests = []
        for r in range(2):
            k = q * 2 + r
            koff = k * _LANES
            bm = gbufm[pl.ds(koff, _LANES)]
            bi = gbufi[pl.ds(koff, _LANES)]
            for j in range(1, _WPG):
                vm = gbufm[pl.ds(j * sz + koff, _LANES)]
                vi = gbufi[pl.ds(j * sz + koff, _LANES)]
                bm, bi = _lex_merge(bm, bi, vm, vi)
            best_v = bm[0]
            best = bi[0]
            for j in range(1, _LANES):
                v = bm[j]
                ij = bi[j]
                take = (v > best_v) | ((v == best_v) & (ij < best))
                best_v = jnp.where(take, v, best_v)
                best = jnp.where(take, ij, best)
            bests.append(best)
        vout = jnp.where(iota == 0, bests[0], jnp.where(iota == 1, bests[1], 0))
        obuf[...] = vout
        slot = (row8 + q * 2) // 2
        dst = pl.multiple_of(slot * _LANES, 8)
        pltpu.sync_copy(obuf, idcs_hbm.at[pl.ds(dst, _LANES)])

    return scan_argmax


def _make_gather_kernel(S, N, d, n_workers):
    rows_per_w = S // n_workers
    mesh = plsc.VectorSubcoreMesh(core_axis_name="c", subcore_axis_name="s")

    @functools.partial(
        pl.kernel,
        out_type=jax.ShapeDtypeStruct((S * d,), jnp.float32),
        mesh=mesh,
        compiler_params=pltpu.CompilerParams(needs_layout_passes=False),
        scratch_types=[
            pltpu.VMEM((S * _LANES // 2,), jnp.int32),
            pltpu.VMEM((d, 128), jnp.float32),
            pltpu.VMEM((d,), jnp.float32),
        ],
    )
    def gather_rows(xt_hbm, idcs_hbm, out_hbm, ibuf, xblk, row_v):
        cid = lax.axis_index("c")
        sid = lax.axis_index("s")
        wid = sid * 2 + cid
        iota = lax.iota(jnp.int32, _LANES)

        pltpu.sync_copy(idcs_hbm, ibuf)
        lane0 = ibuf[pl.ds(wid * _LANES, _LANES)]
        for r in range(rows_per_w):
            best = lane0[r]
            base = pl.multiple_of((best // 128) * 128, 128)
            col = best - base
            pltpu.sync_copy(xt_hbm.at[pl.ds(0, d), pl.ds(base, 128)], xblk)
            colv = jnp.zeros((_LANES,), jnp.int32) + col
            for k in range(d // _LANES):
                vals = plsc.load_gather(xblk, [iota + k * _LANES, colv])
                row_v[pl.ds(k * _LANES, _LANES)] = vals
            row = wid * rows_per_w + r
            pltpu.sync_copy(
                row_v, out_hbm.at[pl.ds(pl.multiple_of(row * d, 8), d)]
            )

    return gather_rows


def kernel(X, samples, num_samples):
    S, N, _ = samples.shape
    d = X.shape[1]
    info = plsc.get_sparse_core_info()
    n_workers = info.num_cores * info.num_subcores
    obj = jnp.squeeze(samples, axis=-1)
    xt = jnp.transpose(X)
    tail = jnp.pad(
        lax.slice(obj, (0, _TAIL_OFF), (S, N)), ((0, 0), (0, 256 - _TAIL))
    )
    idcs = _make_argmax_kernel(S, N, n_workers)(obj, tail)
    out = _make_gather_kernel(S, N, d, n_workers)(xt, idcs)
    return out.reshape(S, d)
